# baseline (speedup 1.0000x reference)
import functools
import math

import jax
import jax.numpy as jnp
import numpy as _np
from jax.experimental import pallas as pl
from jax.experimental.pallas import tpu as pltpu

DIM = 96
NWIN = 14
R = 16
L = R * R
NREG = NWIN * NWIN
TOPK = 4
NH = 8
HD = DIM // NH
SEQ = 4
H = W = NWIN * R

_FW = [float(_np.float32(math.exp(2.0 - SEQ - t))) for t in range(SEQ - 1)]


def _prep_kernel(x_ref, qw_ref, qb_ref, kvw_ref, kvb_ref,
                 q_ref, k_ref, v_ref, qm_ref, km_ref):
    x4 = x_ref[...]
    his3 = x4[0] * _FW[0] + x4[1] * _FW[1] + x4[2] * _FW[2]
    now3 = x4[SEQ - 1]
    q = jax.lax.dot_general(now3, qw_ref[...], (((0,), (1,)), ((), ())),
                            preferred_element_type=jnp.float32
                            ).reshape(R * W, DIM) + qb_ref[...]
    kv = jax.lax.dot_general(his3, kvw_ref[...], (((0,), (1,)), ((), ())),
                             preferred_element_type=jnp.float32
                             ).reshape(R * W, 2 * DIM) + kvb_ref[...]
    k = kv[:, :DIM]
    v = kv[:, DIM:]
    q4 = q.reshape(R, NWIN, R, DIM)
    k4 = k.reshape(R, NWIN, R, DIM)
    qm_ref[...] = jnp.mean(q4, axis=(0, 2)).reshape(1, NWIN, DIM)
    km_ref[...] = jnp.mean(k4, axis=(0, 2)).reshape(1, NWIN, DIM)
    q_ref[...] = (q * (DIM ** -0.5)).astype(jnp.bfloat16).reshape(R, W, DIM)
    k_ref[...] = k.astype(jnp.bfloat16).reshape(R, W, DIM)
    v_ref[...] = v.astype(jnp.bfloat16).reshape(R, W, DIM)


def _route_kernel(qm_ref, km_ref, idx_ref):
    a = jax.lax.dot_general(qm_ref[...], km_ref[...], (((1,), (1,)), ((), ())),
                            preferred_element_type=jnp.float32)
    iota = jax.lax.broadcasted_iota(jnp.int32, (NREG, NREG), 1)
    cols = []
    for _ in range(TOPK):
        m = jnp.max(a, axis=1, keepdims=True)
        sel = a >= m
        pick = jnp.min(jnp.where(sel, iota, NREG), axis=1, keepdims=True)
        cols.append(pick)
        a = jnp.where(iota == pick, -jnp.inf, a)
    idx_ref[...] = jnp.concatenate(cols, axis=1)


def _attn_kernel(idx_ref, q_ref, k0, k1, k2, k3, v0, v1, v2, v3, o_ref):
    del idx_ref
    q = q_ref[...].reshape(L, DIM)
    ks = [r[...].reshape(L, DIM) for r in (k0, k1, k2, k3)]
    vs = [r[...].reshape(L, DIM) for r in (v0, v1, v2, v3)]
    ones = jnp.ones((L, 1), jnp.bfloat16)
    outs = []
    for h in range(NH):
        sl = slice(h * HD, (h + 1) * HD)
        qh = q[:, sl]
        sts = [jax.lax.dot_general(kj[:, sl], qh, (((1,), (1,)), ((), ())),
                                   preferred_element_type=jnp.float32
                                   ).astype(jnp.bfloat16)
               for kj in ks]
        mx = functools.reduce(jnp.maximum,
                              [jnp.max(s, axis=0, keepdims=True) for s in sts])
        es = [jnp.exp(s - mx) for s in sts]
        acc = sum(jax.lax.dot_general(
                      jnp.concatenate([vj[:, sl], ones], axis=1), e,
                      (((0,), (0,)), ((), ())),
                      preferred_element_type=jnp.float32)
                  for vj, e in zip(vs, es))
        outs.append(acc[:HD] / acc[HD:])
    o = jnp.concatenate(outs, axis=0)
    o_ref[...] = jnp.transpose(o, (1, 0)).reshape(R, R, DIM)


def _final_kernel(off_ref, at_ref, vu_ref, vc_ref, vd_ref, lw_ref, lb_ref,
                  ow_ref, ob_ref, o_ref):
    i = off_ref[0] + pl.program_id(0)
    vc = vc_ref[...].astype(jnp.float32)
    up = jnp.where(i > 0, vu_ref[R - 1, :, :].astype(jnp.float32), 0.0)
    dn = jnp.where(i < NWIN - 1, vd_ref[0, :, :].astype(jnp.float32), 0.0)
    rows = jnp.concatenate([up[None], vc, dn[None]], axis=0)
    pad = jnp.pad(rows, ((0, 0), (1, 1), (0, 0)))
    lw = lw_ref[...]
    acc = lb_ref[...].reshape(1, 1, DIM)
    acc = acc + sum(
        lw[3 * dy + dx].reshape(1, 1, DIM) * pad[dy:dy + R, dx:dx + W, :]
        for dy in range(3) for dx in range(3))
    y = (at_ref[...] + acc).reshape(R * W, DIM)
    out = jax.lax.dot_general(ow_ref[...], y, (((1,), (1,)), ((), ())),
                              preferred_element_type=jnp.float32)
    out = out + ob_ref[...].reshape(DIM, 1)
    o_ref[...] = out.reshape(DIM, R, W)


def _device_kernel(nsh, x, q_w, q_b, kv_w, kv_b, out_w, out_b, lw9, lepe_b):
    f32 = jnp.float32
    bf16 = jnp.bfloat16
    nwin_loc = NWIN // nsh
    nreg_loc = NREG // nsh
    h_loc = H // nsh
    d = jax.lax.axis_index("d") if nsh > 1 else 0

    q_loc, k_loc, v_loc, qm_loc, km_loc = pl.pallas_call(
        _prep_kernel,
        grid=(nwin_loc,),
        in_specs=[
            pl.BlockSpec((SEQ, DIM, R, W), lambda i: (0, 0, i, 0)),
            pl.BlockSpec((DIM, DIM), lambda i: (0, 0)),
            pl.BlockSpec((DIM,), lambda i: (0,)),
            pl.BlockSpec((2 * DIM, DIM), lambda i: (0, 0)),
            pl.BlockSpec((2 * DIM,), lambda i: (0,)),
        ],
        out_specs=[
            pl.BlockSpec((R, W, DIM), lambda i: (i, 0, 0)),
            pl.BlockSpec((R, W, DIM), lambda i: (i, 0, 0)),
            pl.BlockSpec((R, W, DIM), lambda i: (i, 0, 0)),
            pl.BlockSpec((1, NWIN, DIM), lambda i: (i, 0, 0)),
            pl.BlockSpec((1, NWIN, DIM), lambda i: (i, 0, 0)),
        ],
        out_shape=[
            jax.ShapeDtypeStruct((h_loc, W, DIM), bf16),
            jax.ShapeDtypeStruct((h_loc, W, DIM), bf16),
            jax.ShapeDtypeStruct((h_loc, W, DIM), bf16),
            jax.ShapeDtypeStruct((nwin_loc, NWIN, DIM), f32),
            jax.ShapeDtypeStruct((nwin_loc, NWIN, DIM), f32),
        ],
    )(x, q_w, q_b, kv_w, kv_b)

    if nsh > 1:
        k_full = jax.lax.all_gather(k_loc, "d", axis=0, tiled=True)
        v_full = jax.lax.all_gather(v_loc, "d", axis=0, tiled=True)
        q_mean = jax.lax.all_gather(qm_loc, "d", axis=0, tiled=True)
        k_mean = jax.lax.all_gather(km_loc, "d", axis=0, tiled=True)
    else:
        k_full, v_full, q_mean, k_mean = k_loc, v_loc, qm_loc, km_loc

    idx = pl.pallas_call(
        _route_kernel,
        out_shape=jax.ShapeDtypeStruct((NREG, TOPK), jnp.int32),
    )(q_mean.reshape(NREG, DIM), k_mean.reshape(NREG, DIM))
    idx_loc = jax.lax.dynamic_slice(idx, (d * nreg_loc, 0), (nreg_loc, TOPK))

    def _kv_spec(j):
        return pl.BlockSpec(
            (R, R, DIM),
            lambda n, idx_ref, j=j: (idx_ref[n, j] // NWIN,
                                     idx_ref[n, j] % NWIN, 0))

    attn = pl.pallas_call(
        _attn_kernel,
        grid_spec=pltpu.PrefetchScalarGridSpec(
            num_scalar_prefetch=1,
            grid=(nreg_loc,),
            in_specs=[pl.BlockSpec((R, R, DIM),
                                   lambda n, idx_ref: (n // NWIN, n % NWIN, 0))]
                     + [_kv_spec(j) for j in range(TOPK)] * 2,
            out_specs=pl.BlockSpec((R, R, DIM),
                                   lambda n, idx_ref: (n // NWIN, n % NWIN, 0)),
        ),
        out_shape=jax.ShapeDtypeStruct((h_loc, W, DIM), f32),
    )(idx_loc, q_loc, k_full, k_full, k_full, k_full,
      v_full, v_full, v_full, v_full)

    off = jnp.full((1,), d * nwin_loc, jnp.int32)

    def row(dr):
        return lambda i, off_ref: (
            jnp.clip(off_ref[0] + i + dr, 0, NWIN - 1), 0, 0)

    out = pl.pallas_call(
        _final_kernel,
        grid_spec=pltpu.PrefetchScalarGridSpec(
            num_scalar_prefetch=1,
            grid=(nwin_loc,),
            in_specs=[
                pl.BlockSpec((R, W, DIM), lambda i, off_ref: (i, 0, 0)),
                pl.BlockSpec((R, W, DIM), row(-1)),
                pl.BlockSpec((R, W, DIM), row(0)),
                pl.BlockSpec((R, W, DIM), row(1)),
                pl.BlockSpec((9, DIM), lambda i, off_ref: (0, 0)),
                pl.BlockSpec((DIM,), lambda i, off_ref: (0,)),
                pl.BlockSpec((DIM, DIM), lambda i, off_ref: (0, 0)),
                pl.BlockSpec((DIM,), lambda i, off_ref: (0,)),
            ],
            out_specs=pl.BlockSpec((DIM, R, W),
                                   lambda i, off_ref: (0, i, 0)),
        ),
        out_shape=jax.ShapeDtypeStruct((DIM, h_loc, W), f32),
    )(off, attn, v_full, v_full, v_full, lw9, lepe_b, out_w, out_b)

    return out


def kernel(x, q_w, q_b, kv_w, kv_b, out_w, out_b, lepe_w, lepe_b):
    devs = jax.devices()
    nsh = 2 if len(devs) >= 2 else 1
    x4 = x.reshape(SEQ, DIM, H, W)
    lw9 = lepe_w.reshape(DIM, 9).T
    if nsh == 1:
        out = _device_kernel(1, x4, q_w, q_b, kv_w, kv_b, out_w, out_b,
                             lw9, lepe_b)
    else:
        mesh = jax.make_mesh((nsh,), ("d",), devices=devs[:nsh])
        P = jax.sharding.PartitionSpec
        x4 = jax.reshard(
            x4, jax.sharding.NamedSharding(mesh, P(None, None, "d", None)))
        out = jax.shard_map(
            functools.partial(_device_kernel, nsh),
            mesh=mesh,
            in_specs=(P(None, None, "d", None), P(None, None), P(None),
                      P(None, None), P(None), P(None, None), P(None),
                      P(None, None), P(None)),
            out_specs=P(None, "d", None),
            check_vma=False,
        )(x4, q_w, q_b, kv_w, kv_b, out_w, out_b, lw9, lepe_b)
    return out.reshape(1, DIM, H, W)


# speedup vs baseline: 1.1312x; 1.1312x over previous
import functools
import math

import jax
import jax.numpy as jnp
import numpy as _np
from jax.experimental import pallas as pl
from jax.experimental.pallas import tpu as pltpu

DIM = 96
NWIN = 14
R = 16
L = R * R
NREG = NWIN * NWIN
TOPK = 4
NH = 8
HD = DIM // NH
SEQ = 4
H = W = NWIN * R

_FW = [float(_np.float32(math.exp(2.0 - SEQ - t))) for t in range(SEQ - 1)]


def _prep_kernel(x_ref, qw_ref, qb_ref, kvw_ref, kvb_ref,
                 q_ref, k_ref, v_ref, qm_ref, km_ref):
    x4 = x_ref[...]
    his3 = x4[0] * _FW[0] + x4[1] * _FW[1] + x4[2] * _FW[2]
    now3 = x4[SEQ - 1]
    q = jax.lax.dot_general(now3, qw_ref[...], (((0,), (1,)), ((), ())),
                            preferred_element_type=jnp.float32
                            ).reshape(R * W, DIM) + qb_ref[...]
    kv = jax.lax.dot_general(his3, kvw_ref[...], (((0,), (1,)), ((), ())),
                             preferred_element_type=jnp.float32
                             ).reshape(R * W, 2 * DIM) + kvb_ref[...]
    k = kv[:, :DIM]
    v = kv[:, DIM:]
    q4 = q.reshape(R, NWIN, R, DIM)
    k4 = k.reshape(R, NWIN, R, DIM)
    qm_ref[...] = jnp.mean(q4, axis=(0, 2)).reshape(1, NWIN, DIM)
    km_ref[...] = jnp.mean(k4, axis=(0, 2)).reshape(1, NWIN, DIM)
    q_ref[...] = (q * (DIM ** -0.5)).astype(jnp.bfloat16).reshape(R, W, DIM)
    k_ref[...] = k.astype(jnp.bfloat16).reshape(R, W, DIM)
    v_ref[...] = v.astype(jnp.bfloat16).reshape(R, W, DIM)


def _route_kernel(qm_ref, km_ref, idx_ref):
    a = jax.lax.dot_general(qm_ref[...], km_ref[...], (((1,), (1,)), ((), ())),
                            preferred_element_type=jnp.float32)
    iota = jax.lax.broadcasted_iota(jnp.int32, (NREG, NREG), 1)
    cols = []
    for _ in range(TOPK):
        m = jnp.max(a, axis=1, keepdims=True)
        sel = a >= m
        pick = jnp.min(jnp.where(sel, iota, NREG), axis=1, keepdims=True)
        cols.append(pick)
        a = jnp.where(iota == pick, -jnp.inf, a)
    idx_ref[...] = jnp.concatenate(cols, axis=1)


def _attn_kernel(idx_ref, q_ref, k0, k1, k2, k3, v0, v1, v2, v3, o_ref):
    del idx_ref
    q = q_ref[...].reshape(L, DIM)
    ks = [r[...].reshape(L, DIM) for r in (k0, k1, k2, k3)]
    vs = [r[...].reshape(L, DIM) for r in (v0, v1, v2, v3)]
    ones = jnp.ones((L, 1), jnp.bfloat16)
    outs = []
    for h in range(NH):
        sl = slice(h * HD, (h + 1) * HD)
        qh = q[:, sl]
        sts = [jax.lax.dot_general(kj[:, sl], qh, (((1,), (1,)), ((), ())),
                                   preferred_element_type=jnp.float32
                                   ).astype(jnp.bfloat16)
               for kj in ks]
        mx = functools.reduce(jnp.maximum,
                              [jnp.max(s, axis=0, keepdims=True) for s in sts])
        es = [jnp.exp(s - mx) for s in sts]
        acc = sum(jax.lax.dot_general(
                      jnp.concatenate([vj[:, sl], ones], axis=1), e,
                      (((0,), (0,)), ((), ())),
                      preferred_element_type=jnp.float32)
                  for vj, e in zip(vs, es))
        outs.append(acc[:HD] / acc[HD:])
    o = jnp.concatenate(outs, axis=0)
    o_ref[...] = jnp.transpose(o, (1, 0)).reshape(R, R, DIM)


def _final_kernel(off_ref, at_ref, vu_ref, vc_ref, vd_ref, lw_ref, lb_ref,
                  ow_ref, ob_ref, o_ref):
    i = off_ref[0] + pl.program_id(0)
    vc = vc_ref[...].astype(jnp.float32)
    up = jnp.where(i > 0, vu_ref[R - 1, :, :].astype(jnp.float32), 0.0)
    dn = jnp.where(i < NWIN - 1, vd_ref[0, :, :].astype(jnp.float32), 0.0)
    rows = jnp.concatenate([up[None], vc, dn[None]], axis=0)
    pad = jnp.pad(rows, ((0, 0), (1, 1), (0, 0)))
    lw = lw_ref[...]
    acc = lb_ref[...].reshape(1, 1, DIM)
    acc = acc + sum(
        lw[3 * dy + dx].reshape(1, 1, DIM) * pad[dy:dy + R, dx:dx + W, :]
        for dy in range(3) for dx in range(3))
    y = (at_ref[...] + acc).reshape(R * W, DIM)
    out = jax.lax.dot_general(ow_ref[...], y, (((1,), (1,)), ((), ())),
                              preferred_element_type=jnp.float32)
    out = out + ob_ref[...].reshape(DIM, 1)
    o_ref[...] = out.reshape(DIM, R, W)


def _device_kernel(nsh, x, q_w, q_b, kv_w, kv_b, out_w, out_b, lw9, lepe_b):
    f32 = jnp.float32
    bf16 = jnp.bfloat16
    nwin_loc = NWIN // nsh
    nreg_loc = NREG // nsh
    h_loc = H // nsh
    d = jax.lax.axis_index("d") if nsh > 1 else 0

    q_loc, k_loc, v_loc, qm_loc, km_loc = pl.pallas_call(
        _prep_kernel,
        grid=(nwin_loc,),
        in_specs=[
            pl.BlockSpec((SEQ, DIM, R, W), lambda i: (0, 0, i, 0)),
            pl.BlockSpec((DIM, DIM), lambda i: (0, 0)),
            pl.BlockSpec((DIM,), lambda i: (0,)),
            pl.BlockSpec((2 * DIM, DIM), lambda i: (0, 0)),
            pl.BlockSpec((2 * DIM,), lambda i: (0,)),
        ],
        out_specs=[
            pl.BlockSpec((R, W, DIM), lambda i: (i, 0, 0)),
            pl.BlockSpec((R, W, DIM), lambda i: (i, 0, 0)),
            pl.BlockSpec((R, W, DIM), lambda i: (i, 0, 0)),
            pl.BlockSpec((1, NWIN, DIM), lambda i: (i, 0, 0)),
            pl.BlockSpec((1, NWIN, DIM), lambda i: (i, 0, 0)),
        ],
        out_shape=[
            jax.ShapeDtypeStruct((h_loc, W, DIM), bf16),
            jax.ShapeDtypeStruct((h_loc, W, DIM), bf16),
            jax.ShapeDtypeStruct((h_loc, W, DIM), bf16),
            jax.ShapeDtypeStruct((nwin_loc, NWIN, DIM), f32),
            jax.ShapeDtypeStruct((nwin_loc, NWIN, DIM), f32),
        ],
    )(x, q_w, q_b, kv_w, kv_b)

    if nsh > 1:
        k_full = jax.lax.all_gather(k_loc, "d", axis=0, tiled=True)
        v_full = jax.lax.all_gather(v_loc, "d", axis=0, tiled=True)
        q_mean = jax.lax.all_gather(qm_loc, "d", axis=0, tiled=True)
        k_mean = jax.lax.all_gather(km_loc, "d", axis=0, tiled=True)
    else:
        k_full, v_full, q_mean, k_mean = k_loc, v_loc, qm_loc, km_loc

    idx = pl.pallas_call(
        _route_kernel,
        out_shape=jax.ShapeDtypeStruct((NREG, TOPK), jnp.int32),
    )(q_mean.reshape(NREG, DIM), k_mean.reshape(NREG, DIM))
    idx_loc = jax.lax.dynamic_slice(idx, (d * nreg_loc, 0), (nreg_loc, TOPK))

    def _kv_spec(j):
        return pl.BlockSpec(
            (R, R, DIM),
            lambda n, idx_ref, j=j: (idx_ref[n, j] // NWIN,
                                     idx_ref[n, j] % NWIN, 0))

    attn = pl.pallas_call(
        _attn_kernel,
        grid_spec=pltpu.PrefetchScalarGridSpec(
            num_scalar_prefetch=1,
            grid=(nreg_loc,),
            in_specs=[pl.BlockSpec((R, R, DIM),
                                   lambda n, idx_ref: (n // NWIN, n % NWIN, 0))]
                     + [_kv_spec(j) for j in range(TOPK)] * 2,
            out_specs=pl.BlockSpec((R, R, DIM),
                                   lambda n, idx_ref: (n // NWIN, n % NWIN, 0)),
        ),
        out_shape=jax.ShapeDtypeStruct((h_loc, W, DIM), f32),
    )(idx_loc, q_loc, k_full, k_full, k_full, k_full,
      v_full, v_full, v_full, v_full)

    off = jnp.full((1,), d * nwin_loc, jnp.int32)

    def row(dr):
        return lambda i, off_ref: (
            jnp.clip(off_ref[0] + i + dr, 0, NWIN - 1), 0, 0)

    out = pl.pallas_call(
        _final_kernel,
        grid_spec=pltpu.PrefetchScalarGridSpec(
            num_scalar_prefetch=1,
            grid=(nwin_loc,),
            in_specs=[
                pl.BlockSpec((R, W, DIM), lambda i, off_ref: (i, 0, 0)),
                pl.BlockSpec((R, W, DIM), row(-1)),
                pl.BlockSpec((R, W, DIM), row(0)),
                pl.BlockSpec((R, W, DIM), row(1)),
                pl.BlockSpec((9, DIM), lambda i, off_ref: (0, 0)),
                pl.BlockSpec((DIM,), lambda i, off_ref: (0,)),
                pl.BlockSpec((DIM, DIM), lambda i, off_ref: (0, 0)),
                pl.BlockSpec((DIM,), lambda i, off_ref: (0,)),
            ],
            out_specs=pl.BlockSpec((DIM, R, W),
                                   lambda i, off_ref: (0, i, 0)),
        ),
        out_shape=jax.ShapeDtypeStruct((DIM, h_loc, W), f32),
    )(off, attn, v_full, v_full, v_full, lw9, lepe_b, out_w, out_b)

    return out


def kernel(x, q_w, q_b, kv_w, kv_b, out_w, out_b, lepe_w, lepe_b):
    devs = jax.devices()
    nsh = 1
    x4 = x.reshape(SEQ, DIM, H, W)
    lw9 = lepe_w.reshape(DIM, 9).T
    if nsh == 1:
        out = _device_kernel(1, x4, q_w, q_b, kv_w, kv_b, out_w, out_b,
                             lw9, lepe_b)
    else:
        mesh = jax.make_mesh((nsh,), ("d",), devices=devs[:nsh])
        P = jax.sharding.PartitionSpec
        x4 = jax.reshard(
            x4, jax.sharding.NamedSharding(mesh, P(None, None, "d", None)))
        out = jax.shard_map(
            functools.partial(_device_kernel, nsh),
            mesh=mesh,
            in_specs=(P(None, None, "d", None), P(None, None), P(None),
                      P(None, None), P(None), P(None, None), P(None),
                      P(None, None), P(None)),
            out_specs=P(None, "d", None),
            check_vma=False,
        )(x4, q_w, q_b, kv_w, kv_b, out_w, out_b, lw9, lepe_b)
    return out.reshape(1, DIM, H, W)
